# Initial kernel scaffold; baseline (speedup 1.0000x reference)
import jax
import jax.numpy as jnp
from jax.experimental import pallas as pl


def kernel(x, bias, fc_table, emb_table, W0, b0, W1, b1, W2, b2, W3, b3):
    raise NotImplementedError("write your pallas kernel here")


# speedup vs baseline: 1.5748x; 1.5748x over previous
import functools

import jax
import jax.numpy as jnp
from jax import lax
from jax.experimental import pallas as pl
from jax.experimental.pallas import tpu as pltpu
from jax.experimental.pallas import tpu_sc as plsc

V = 201000
D = 128
NF = 3
B = 4096
R = B * NF

_NC, _NS = 2, 16
NW = _NC * _NS
B_PER_W = R // NW

@functools.cache
def _make_sc_gather():
    mesh = plsc.VectorSubcoreMesh(core_axis_name="c", subcore_axis_name="s",
                                  num_cores=_NC, num_subcores=_NS)

    @functools.partial(
        pl.kernel,
        out_type=(
            jax.ShapeDtypeStruct((R, D), jnp.float32),
            jax.ShapeDtypeStruct((R,), jnp.float32),
        ),
        mesh=mesh,
        scratch_types=(
            pltpu.VMEM((B_PER_W,), jnp.int32),
            pltpu.VMEM((B_PER_W, D), jnp.float32),
            pltpu.VMEM((B_PER_W,), jnp.float32),
            pltpu.SemaphoreType.DMA,
            pltpu.SemaphoreType.DMA,
        ),
    )
    def _sc_gather(emb_hbm, fc_hbm, idx_hbm, emb_out, fc_out,
                   idx_v, rows_v, fc_v, sem_e, sem_f):
        wid = lax.axis_index("s") * _NC + lax.axis_index("c")
        base = wid * B_PER_W
        pltpu.sync_copy(idx_hbm.at[pl.ds(base, B_PER_W)], idx_v)
        cp_e = pltpu.async_copy(emb_hbm.at[idx_v], rows_v, sem_e)
        cp_f = pltpu.async_copy(fc_hbm.at[idx_v], fc_v, sem_f)
        cp_e.wait()
        pltpu.sync_copy(rows_v, emb_out.at[pl.ds(base, B_PER_W)])
        cp_f.wait()
        pltpu.sync_copy(fc_v, fc_out.at[pl.ds(base, B_PER_W)])

    return _sc_gather


_BLK = 512
_GRID = B // _BLK


def _dense_body(h_ref, fc_ref, bias_ref, w0_ref, b0_ref, w1_ref, b1_ref,
                w2_ref, b2_ref, w3_ref, b3_ref, out_ref):
    h = h_ref[...]
    e0 = h[:, :D]
    e1 = h[:, D:2 * D]
    e2 = h[:, 2 * D:]
    s = e0 + e1 + e2
    quad = s * s - (e0 * e0 + e1 * e1 + e2 * e2)
    fm = (bias_ref[0]
          + jnp.sum(fc_ref[...], axis=1)
          + 0.5 * jnp.sum(quad, axis=1))
    a = jnp.maximum(jnp.dot(h, w0_ref[...],
                            preferred_element_type=jnp.float32) + b0_ref[...], 0.0)
    a = jnp.maximum(jnp.dot(a, w1_ref[...],
                            preferred_element_type=jnp.float32) + b1_ref[...], 0.0)
    a = jnp.maximum(jnp.dot(a, w2_ref[...],
                            preferred_element_type=jnp.float32) + b2_ref[...], 0.0)
    mlp = jnp.sum(a * w3_ref[...][:, 0][None, :], axis=1) + b3_ref[0]
    out_ref[...] = jax.nn.sigmoid(fm + mlp)


def _dense(h, fc, bias, W0, b0, W1, b1, W2, b2, W3, b3):
    full = lambda shape: pl.BlockSpec(shape, lambda i: (0,) * len(shape))
    return pl.pallas_call(
        _dense_body,
        grid=(_GRID,),
        in_specs=[
            pl.BlockSpec((_BLK, NF * D), lambda i: (i, 0)),
            pl.BlockSpec((_BLK, NF), lambda i: (i, 0)),
            full((1,)),
            full((NF * D, 256)), full((256,)),
            full((256, 128)), full((128,)),
            full((128, 64)), full((64,)),
            full((64, 1)), full((1,)),
        ],
        out_specs=pl.BlockSpec((_BLK,), lambda i: (i,)),
        out_shape=jax.ShapeDtypeStruct((B,), jnp.float32),
    )(h, fc, bias, W0, b0, W1, b1, W2, b2, W3, b3)


def kernel(x, bias, fc_table, emb_table, W0, b0, W1, b1, W2, b2, W3, b3):
    idx = x.reshape(-1).astype(jnp.int32)
    emb_rows, fc_vals = _make_sc_gather()(emb_table, fc_table.reshape(-1), idx)
    h = emb_rows.reshape(B, NF * D)
    fc = fc_vals.reshape(B, NF)
    return _dense(h, fc, bias, W0, b0, W1, b1, W2, b2, W3, b3)
